# baseline (speedup 1.0000x reference)
import functools
import math

import jax
import jax.numpy as jnp
from jax import lax
from jax.experimental import pallas as pl
from jax.experimental.pallas import tpu as pltpu
from jax.experimental.pallas import tpu_sc as plsc

TOPK = 8

SC_CORES = 2
SC_SUBCORES = 16
SC_LANES = 16


def _fold_body(wq_ref, wr_ref, ak_ref, r1_ref, rk_ref):
    r1_ref[...] = lax.dot_general(
        wr_ref[...], wq_ref[...], (((1,), (0,)), ((), ())),
        preferred_element_type=jnp.float32).astype(jnp.bfloat16)
    rk_ref[...] = lax.dot_general(
        ak_ref[...], wr_ref[...], (((1,), (1,)), ((), ())),
        preferred_element_type=jnp.float32).astype(jnp.bfloat16)


def _fold_weights(Wq, Wr, aux_keys):
    rd, bottle = Wr.shape
    hidden = Wq.shape[1]
    slots = aux_keys.shape[0]
    return pl.pallas_call(
        _fold_body,
        out_shape=(
            jax.ShapeDtypeStruct((rd, hidden), jnp.bfloat16),
            jax.ShapeDtypeStruct((slots, rd), jnp.bfloat16),
        ),
    )(Wq, Wr, aux_keys)


def _main_body(hs_ref, r1_ref, rk_ref, mask_ref, uw1_ref, ub1_ref, uw2_ref,
               ub2_ref, w_ref, idx_ref, lv_ref, up_ref, lvs_ref):
    ts, hidden = hs_ref.shape
    slots = rk_ref.shape[0]
    hs = hs_ref[...]
    hs_bf = hs.astype(jnp.bfloat16)

    rq = lax.dot_general(hs_bf, r1_ref[...], (((1,), (1,)), ((), ())),
                         preferred_element_type=jnp.float32)
    scores = lax.dot_general(rq.astype(jnp.bfloat16), rk_ref[...],
                             (((1,), (1,)), ((), ())),
                             preferred_element_type=jnp.float32)
    scores = scores * (1.0 / math.sqrt(rk_ref.shape[1])) + mask_ref[...]

    iota = lax.broadcasted_iota(jnp.int32, (ts, slots), 1)
    sbit = jnp.int32(-2147483648)
    u = lax.bitcast_convert_type(scores, jnp.int32)
    key = jnp.where(u < 0, jnp.bitwise_not(u) ^ sbit, u)
    key = (key & jnp.int32(-slots)) | (slots - 1 - iota)
    vals = []
    idxs = []
    for _ in range(TOPK):
        m = jnp.max(key, axis=1, keepdims=True)
        idxs.append(slots - 1 - (m & (slots - 1)))
        kp = m & jnp.int32(-slots)
        uq = jnp.where(kp < 0, jnp.bitwise_not(kp ^ sbit), kp)
        vals.append(lax.bitcast_convert_type(uq, jnp.float32))
        key = jnp.where(key == m, sbit, key)
    tv = jnp.concatenate(vals, axis=1)
    ti = jnp.concatenate(idxs, axis=1)

    e = jnp.exp(tv - tv[:, 0:1])
    w_ref[...] = e / jnp.sum(e, axis=1, keepdims=True)
    idx_ref[...] = ti

    mu = jnp.mean(hs, axis=1, keepdims=True)
    var = jnp.mean((hs - mu) ** 2, axis=1, keepdims=True)
    lv = jnp.log1p(var)
    lv_ref[...] = lv
    h1 = lax.dot_general(hs_bf, uw1_ref[...], (((1,), (1,)), ((), ())),
                         preferred_element_type=jnp.float32) + ub1_ref[...]
    h1 = jax.nn.gelu(h1)
    learned = jnp.sum(h1 * uw2_ref[...], axis=1, keepdims=True)
    learned = learned + ub2_ref[0, 0]
    up_ref[...] = jax.nn.sigmoid(learned) * 2.5

    @pl.when(pl.program_id(0) == 0)
    def _():
        lvs_ref[0, 0] = 0.0

    lvs_ref[0, 0] += jnp.sum(lv)


def _run_main(hs2, R1, rk, mask, u_w1, u_b1, u_w2, u_b2, ts=512):
    nt, hidden = hs2.shape
    slots, rd = rk.shape
    h4 = u_w1.shape[0]
    grid = (nt // ts,)
    return pl.pallas_call(
        _main_body,
        grid=grid,
        in_specs=[
            pl.BlockSpec((ts, hidden), lambda i: (i, 0)),
            pl.BlockSpec((rd, hidden), lambda i: (0, 0)),
            pl.BlockSpec((slots, rd), lambda i: (0, 0)),
            pl.BlockSpec((1, slots), lambda i: (0, 0)),
            pl.BlockSpec((h4, hidden), lambda i: (0, 0)),
            pl.BlockSpec((1, h4), lambda i: (0, 0)),
            pl.BlockSpec((1, h4), lambda i: (0, 0)),
            pl.BlockSpec((1, 1), lambda i: (0, 0), memory_space=pltpu.SMEM),
        ],
        out_specs=[
            pl.BlockSpec((ts, TOPK), lambda i: (i, 0)),
            pl.BlockSpec((ts, TOPK), lambda i: (i, 0)),
            pl.BlockSpec((ts, 1), lambda i: (i, 0)),
            pl.BlockSpec((ts, 1), lambda i: (i, 0)),
            pl.BlockSpec((1, 1), lambda i: (0, 0), memory_space=pltpu.SMEM),
        ],
        out_shape=(
            jax.ShapeDtypeStruct((nt, TOPK), jnp.float32),
            jax.ShapeDtypeStruct((nt, TOPK), jnp.int32),
            jax.ShapeDtypeStruct((nt, 1), jnp.float32),
            jax.ShapeDtypeStruct((nt, 1), jnp.float32),
            jax.ShapeDtypeStruct((1, 1), jnp.float32),
        ),
    )(hs2, R1, rk, mask, u_w1, u_b1, u_w2, u_b2)


def _make_sc_gather(nt, hidden, slots):
    nw = SC_CORES * SC_SUBCORES
    tpw = nt // nw
    tch = 2
    rpc = tch * TOPK
    nch = tpw // tch
    nbuf = 3
    nmain = (nch // nbuf) * nbuf
    hw = hidden // 2

    mesh = plsc.VectorSubcoreMesh(
        core_axis_name="c", subcore_axis_name="s",
        num_cores=SC_CORES, num_subcores=SC_SUBCORES)

    @functools.partial(
        pl.kernel,
        out_type=jax.ShapeDtypeStruct((nt, hidden), jnp.bfloat16),
        mesh=mesh,
        scratch_types=[
            pltpu.VMEM((tpw * TOPK,), jnp.int32),
            pltpu.VMEM((tpw * TOPK,), jnp.float32),
            pltpu.VMEM((nbuf, rpc, hw), jnp.int32),
            pltpu.VMEM((nbuf, 1, tch * hw), jnp.int32),
        ] + [pltpu.SemaphoreType.DMA] * (2 * nbuf),
        compiler_params=pltpu.CompilerParams(needs_layout_passes=False),
    )
    def sc_gather(idx_hbm, w_hbm, aux_hbm, agg_hbm, idx_v, w_v, rows_v,
                  acc_v, *sems):
        in_sems = sems[:nbuf]
        out_sems = sems[nbuf:]
        wid = lax.axis_index("s") * SC_CORES + lax.axis_index("c")
        tbase = wid * tpw
        pltpu.sync_copy(idx_hbm.at[pl.ds(tbase * TOPK, tpw * TOPK)], idx_v)
        pltpu.sync_copy(w_hbm.at[pl.ds(tbase * TOPK, tpw * TOPK)], w_v)

        def issue_gather(c, b):
            idx_vec = idx_v[pl.ds(c * rpc, rpc)]
            return pltpu.async_copy(aux_hbm.at[idx_vec], rows_v.at[b],
                                    in_sems[b])

        for b in range(nbuf):
            issue_gather(b, b)

        def out_view(c):
            return agg_hbm.at[pl.ds(tbase + c * tch, tch)].bitcast(jnp.int32)

        def wait_out(c, b):
            pltpu.make_async_copy(acc_v.at[b], out_view(c),
                                  out_sems[b]).wait()

        def do_chunk(c, b, guard_out):
            pltpu.make_async_copy(
                aux_hbm.at[idx_v[pl.ds(c * rpc, rpc)]],
                rows_v.at[b], in_sems[b]).wait()

            if guard_out:

                @pl.when(c >= nbuf)
                def _():
                    wait_out(c - nbuf, b)
            else:
                wait_out(c - nbuf, b)

            wchunk = w_v[pl.ds(c * rpc, rpc)]
            for t in range(tch):
                wf = [
                    jnp.full((SC_LANES,), wchunk[t * TOPK + k], jnp.float32)
                    for k in range(TOPK)
                ]

                @pl.loop(0, hw // SC_LANES, unroll=4)
                def _vreg(h, t=t, wf=wf, b=b):
                    sl = pl.ds(h * SC_LANES, SC_LANES)
                    v = rows_v[b, t * TOPK, sl]
                    acc_o = wf[0] * plsc.bitcast(v, jnp.float32)
                    acc_e = wf[0] * plsc.bitcast(
                        lax.shift_left(v, 16), jnp.float32)
                    for k in range(1, TOPK):
                        v = rows_v[b, t * TOPK + k, sl]
                        acc_o = acc_o + wf[k] * plsc.bitcast(
                            v, jnp.float32)
                        acc_e = acc_e + wf[k] * plsc.bitcast(
                            lax.shift_left(v, 16), jnp.float32)
                    word = (
                        (plsc.bitcast(acc_o, jnp.int32)
                         & jnp.int32(-65536))
                        | lax.shift_right_logical(
                            plsc.bitcast(acc_e, jnp.int32), 16))
                    acc_v[b, 0, pl.ds(t * hw + h * SC_LANES,
                                      SC_LANES)] = word

            @pl.when(c + nbuf < nch)
            def _():
                issue_gather(c + nbuf, b)

            pltpu.async_copy(acc_v.at[b], out_view(c), out_sems[b])

        @pl.loop(0, nmain, step=nbuf)
        def _superstep(c0):
            for b in range(nbuf):
                do_chunk(c0 + b, b, guard_out=True)

        for j in range(nch - nmain):
            do_chunk(nmain + j, j, guard_out=False)

        for c in range(nch - nbuf, nch):
            wait_out(c, c % nbuf)

    return sc_gather


def _combine_body(hs_ref, agg_ref, lv_ref, up_ref, lvs_ref, gw_ref, gb_ref,
                  out_ref, *, nt):
    lv_sum = lvs_ref[0, 0]
    for j in range(1, lvs_ref.shape[0]):
        lv_sum += lvs_ref[j, 0]
    mean_lv = lv_sum * (1.0 / nt)
    norm = lv_ref[...] / (mean_lv + 1e-6)
    comb = jnp.clip(norm * 0.5 + up_ref[...], 0.0, 5.0)
    gate = jax.nn.sigmoid(gw_ref[0, 0] * comb + gb_ref[0, 0])
    out_ref[...] = hs_ref[...] + gate * agg_ref[...].astype(jnp.float32)


def _run_combine(hs2, agg, lv, up, lvs, gw, gb, nt_total, ts=512):
    nt, hidden = hs2.shape
    nsl = lvs.shape[0]
    grid = (nt // ts,)
    return pl.pallas_call(
        functools.partial(_combine_body, nt=nt_total),
        grid=grid,
        in_specs=[
            pl.BlockSpec((ts, hidden), lambda i: (i, 0)),
            pl.BlockSpec((ts, hidden), lambda i: (i, 0)),
            pl.BlockSpec((ts, 1), lambda i: (i, 0)),
            pl.BlockSpec((ts, 1), lambda i: (i, 0)),
            pl.BlockSpec((nsl, 1), lambda i: (0, 0), memory_space=pltpu.SMEM),
            pl.BlockSpec((1, 1), lambda i: (0, 0), memory_space=pltpu.SMEM),
            pl.BlockSpec((1, 1), lambda i: (0, 0), memory_space=pltpu.SMEM),
        ],
        out_specs=pl.BlockSpec((ts, hidden), lambda i: (i, 0)),
        out_shape=jax.ShapeDtypeStruct((nt, hidden), jnp.float32),
    )(hs2, agg, lv, up, lvs, gw, gb)


NSLICE = 4


def kernel(hidden_states, reliability_mask, Wq, Wr, aux_keys, aux_values,
           u_w1, u_b1, u_w2, u_b2, gate_w1, gate_bias):
    b, s, hidden = hidden_states.shape
    slots = aux_keys.shape[0]
    nt = b * s
    nts = nt // NSLICE
    hs2 = hidden_states.reshape(nt, hidden)

    R1, rk = _fold_weights(Wq, Wr, aux_keys)
    u_w1_bf = u_w1.astype(jnp.bfloat16)
    mask2 = reliability_mask.reshape(1, slots)
    ub1 = u_b1.reshape(1, -1)
    ub2 = u_b2.reshape(1, 1)
    aux_i32 = lax.bitcast_convert_type(
        aux_values.astype(jnp.bfloat16).reshape(slots, hidden // 2, 2),
        jnp.int32)
    sc_gather = _make_sc_gather(nts, hidden, slots)

    parts = []
    for i in range(NSLICE):
        hs_i = hs2[i * nts:(i + 1) * nts]
        w, idx, lv, up, lvs = _run_main(
            hs_i, R1, rk, mask2, u_w1_bf, ub1, u_w2, ub2)
        agg = sc_gather(idx.reshape(nts * TOPK), w.reshape(nts * TOPK),
                        aux_i32)
        parts.append((hs_i, agg, lv, up, lvs))

    lvs_all = jnp.concatenate([p[4] for p in parts], axis=0)
    gw = gate_w1.reshape(1, 1)
    gb = gate_bias.reshape(1, 1)
    outs = [
        _run_combine(hs_i, agg, lv, up, lvs_all, gw, gb, nt)
        for hs_i, agg, lv, up, _ in parts
    ]
    return jnp.concatenate(outs, axis=0).reshape(b, s, hidden)


# speedup vs baseline: 1.0610x; 1.0610x over previous
import functools
import math

import jax
import jax.numpy as jnp
from jax import lax
from jax.experimental import pallas as pl
from jax.experimental.pallas import tpu as pltpu
from jax.experimental.pallas import tpu_sc as plsc

TOPK = 8

SC_CORES = 2
SC_SUBCORES = 16
SC_LANES = 16


def _fold_body(wq_ref, wr_ref, ak_ref, r1_ref, rk_ref):
    r1_ref[...] = lax.dot_general(
        wr_ref[...], wq_ref[...], (((1,), (0,)), ((), ())),
        preferred_element_type=jnp.float32).astype(jnp.bfloat16)
    rk_ref[...] = lax.dot_general(
        ak_ref[...], wr_ref[...], (((1,), (1,)), ((), ())),
        preferred_element_type=jnp.float32).astype(jnp.bfloat16)


def _fold_weights(Wq, Wr, aux_keys):
    rd, bottle = Wr.shape
    hidden = Wq.shape[1]
    slots = aux_keys.shape[0]
    return pl.pallas_call(
        _fold_body,
        out_shape=(
            jax.ShapeDtypeStruct((rd, hidden), jnp.bfloat16),
            jax.ShapeDtypeStruct((slots, rd), jnp.bfloat16),
        ),
    )(Wq, Wr, aux_keys)


def _main_body(hs_ref, r1_ref, rk_ref, mask_ref, uw1_ref, ub1_ref, uw2_ref,
               ub2_ref, w_ref, idx_ref, lv_ref, up_ref, lvs_ref):
    ts, hidden = hs_ref.shape
    slots = rk_ref.shape[0]
    hs = hs_ref[...]
    hs_bf = hs.astype(jnp.bfloat16)

    rq = lax.dot_general(hs_bf, r1_ref[...], (((1,), (1,)), ((), ())),
                         preferred_element_type=jnp.float32)
    scores = lax.dot_general(rq.astype(jnp.bfloat16), rk_ref[...],
                             (((1,), (1,)), ((), ())),
                             preferred_element_type=jnp.float32)
    scores = scores * (1.0 / math.sqrt(rk_ref.shape[1])) + mask_ref[...]

    iota = lax.broadcasted_iota(jnp.int32, (ts, slots), 1)
    sbit = jnp.int32(-2147483648)
    u = lax.bitcast_convert_type(scores, jnp.int32)
    key = jnp.where(u < 0, jnp.bitwise_not(u) ^ sbit, u)
    key = (key & jnp.int32(-slots)) | (slots - 1 - iota)
    vals = []
    idxs = []
    for _ in range(TOPK):
        m = jnp.max(key, axis=1, keepdims=True)
        idxs.append(slots - 1 - (m & (slots - 1)))
        kp = m & jnp.int32(-slots)
        uq = jnp.where(kp < 0, jnp.bitwise_not(kp ^ sbit), kp)
        vals.append(lax.bitcast_convert_type(uq, jnp.float32))
        key = jnp.where(key == m, sbit, key)
    tv = jnp.concatenate(vals, axis=1)
    ti = jnp.concatenate(idxs, axis=1)

    e = jnp.exp(tv - tv[:, 0:1])
    w_ref[...] = e / jnp.sum(e, axis=1, keepdims=True)
    idx_ref[...] = ti

    mu = jnp.mean(hs, axis=1, keepdims=True)
    var = jnp.mean((hs - mu) ** 2, axis=1, keepdims=True)
    lv = jnp.log1p(var)
    lv_ref[...] = lv
    h1 = lax.dot_general(hs_bf, uw1_ref[...], (((1,), (1,)), ((), ())),
                         preferred_element_type=jnp.float32) + ub1_ref[...]
    h1 = jax.nn.gelu(h1)
    learned = jnp.sum(h1 * uw2_ref[...], axis=1, keepdims=True)
    learned = learned + ub2_ref[0, 0]
    up_ref[...] = jax.nn.sigmoid(learned) * 2.5

    @pl.when(pl.program_id(0) == 0)
    def _():
        lvs_ref[0, 0] = 0.0

    lvs_ref[0, 0] += jnp.sum(lv)


def _run_main(hs2, R1, rk, mask, u_w1, u_b1, u_w2, u_b2, ts=512):
    nt, hidden = hs2.shape
    slots, rd = rk.shape
    h4 = u_w1.shape[0]
    grid = (nt // ts,)
    return pl.pallas_call(
        _main_body,
        grid=grid,
        in_specs=[
            pl.BlockSpec((ts, hidden), lambda i: (i, 0)),
            pl.BlockSpec((rd, hidden), lambda i: (0, 0)),
            pl.BlockSpec((slots, rd), lambda i: (0, 0)),
            pl.BlockSpec((1, slots), lambda i: (0, 0)),
            pl.BlockSpec((h4, hidden), lambda i: (0, 0)),
            pl.BlockSpec((1, h4), lambda i: (0, 0)),
            pl.BlockSpec((1, h4), lambda i: (0, 0)),
            pl.BlockSpec((1, 1), lambda i: (0, 0), memory_space=pltpu.SMEM),
        ],
        out_specs=[
            pl.BlockSpec((ts, TOPK), lambda i: (i, 0)),
            pl.BlockSpec((ts, TOPK), lambda i: (i, 0)),
            pl.BlockSpec((ts, 1), lambda i: (i, 0)),
            pl.BlockSpec((ts, 1), lambda i: (i, 0)),
            pl.BlockSpec((1, 1), lambda i: (0, 0), memory_space=pltpu.SMEM),
        ],
        out_shape=(
            jax.ShapeDtypeStruct((nt, TOPK), jnp.float32),
            jax.ShapeDtypeStruct((nt, TOPK), jnp.int32),
            jax.ShapeDtypeStruct((nt, 1), jnp.float32),
            jax.ShapeDtypeStruct((nt, 1), jnp.float32),
            jax.ShapeDtypeStruct((1, 1), jnp.float32),
        ),
    )(hs2, R1, rk, mask, u_w1, u_b1, u_w2, u_b2)


def _make_sc_gather(nt, hidden, slots):
    nw = SC_CORES * SC_SUBCORES
    tpw = nt // nw
    tch = 2
    rpc = tch * TOPK
    nch = tpw // tch
    nbuf = 3
    nmain = (nch // nbuf) * nbuf
    hw = hidden // 2

    mesh = plsc.VectorSubcoreMesh(
        core_axis_name="c", subcore_axis_name="s",
        num_cores=SC_CORES, num_subcores=SC_SUBCORES)

    @functools.partial(
        pl.kernel,
        out_type=jax.ShapeDtypeStruct((nt, hidden), jnp.bfloat16),
        mesh=mesh,
        scratch_types=[
            pltpu.VMEM((tpw * TOPK,), jnp.int32),
            pltpu.VMEM((tpw * TOPK,), jnp.float32),
            pltpu.VMEM((nbuf, rpc, hw), jnp.int32),
            pltpu.VMEM((nbuf, 1, tch * hw), jnp.int32),
        ] + [pltpu.SemaphoreType.DMA] * (2 * nbuf),
        compiler_params=pltpu.CompilerParams(needs_layout_passes=False),
    )
    def sc_gather(idx_hbm, w_hbm, aux_hbm, agg_hbm, idx_v, w_v, rows_v,
                  acc_v, *sems):
        in_sems = sems[:nbuf]
        out_sems = sems[nbuf:]
        wid = lax.axis_index("s") * SC_CORES + lax.axis_index("c")
        tbase = wid * tpw
        pltpu.sync_copy(idx_hbm.at[pl.ds(tbase * TOPK, tpw * TOPK)], idx_v)
        pltpu.sync_copy(w_hbm.at[pl.ds(tbase * TOPK, tpw * TOPK)], w_v)

        def issue_gather(c, b):
            idx_vec = idx_v[pl.ds(c * rpc, rpc)]
            return pltpu.async_copy(aux_hbm.at[idx_vec], rows_v.at[b],
                                    in_sems[b])

        for b in range(nbuf):
            issue_gather(b, b)

        def out_view(c):
            return agg_hbm.at[pl.ds(tbase + c * tch, tch)].bitcast(jnp.int32)

        def wait_out(c, b):
            pltpu.make_async_copy(acc_v.at[b], out_view(c),
                                  out_sems[b]).wait()

        def do_chunk(c, b, guard_out):
            pltpu.make_async_copy(
                aux_hbm.at[idx_v[pl.ds(c * rpc, rpc)]],
                rows_v.at[b], in_sems[b]).wait()

            if guard_out:

                @pl.when(c >= nbuf)
                def _():
                    wait_out(c - nbuf, b)
            else:
                wait_out(c - nbuf, b)

            wchunk = w_v[pl.ds(c * rpc, rpc)]
            for t in range(tch):
                wf = [
                    jnp.full((SC_LANES,), wchunk[t * TOPK + k], jnp.float32)
                    for k in range(TOPK)
                ]

                @pl.loop(0, hw // SC_LANES, unroll=4)
                def _vreg(h, t=t, wf=wf, b=b):
                    sl = pl.ds(h * SC_LANES, SC_LANES)
                    v = rows_v[b, t * TOPK, sl]
                    acc_o = wf[0] * plsc.bitcast(v, jnp.float32)
                    acc_e = wf[0] * plsc.bitcast(
                        lax.shift_left(v, 16), jnp.float32)
                    for k in range(1, TOPK):
                        v = rows_v[b, t * TOPK + k, sl]
                        acc_o = acc_o + wf[k] * plsc.bitcast(
                            v, jnp.float32)
                        acc_e = acc_e + wf[k] * plsc.bitcast(
                            lax.shift_left(v, 16), jnp.float32)
                    word = (
                        (plsc.bitcast(acc_o, jnp.int32)
                         & jnp.int32(-65536))
                        | lax.shift_right_logical(
                            plsc.bitcast(acc_e, jnp.int32), 16))
                    acc_v[b, 0, pl.ds(t * hw + h * SC_LANES,
                                      SC_LANES)] = word

            @pl.when(c + nbuf < nch)
            def _():
                issue_gather(c + nbuf, b)

            pltpu.async_copy(acc_v.at[b], out_view(c), out_sems[b])

        @pl.loop(0, nmain, step=nbuf)
        def _superstep(c0):
            for b in range(nbuf):
                do_chunk(c0 + b, b, guard_out=True)

        for j in range(nch - nmain):
            do_chunk(nmain + j, j, guard_out=False)

        for c in range(nch - nbuf, nch):
            wait_out(c, c % nbuf)

    return sc_gather


def _combine_body(hs_ref, agg_ref, lv_ref, up_ref, lvs_ref, gw_ref, gb_ref,
                  out_ref, *, nt):
    lv_sum = lvs_ref[0, 0]
    for j in range(1, lvs_ref.shape[0]):
        lv_sum += lvs_ref[j, 0]
    mean_lv = lv_sum * (1.0 / nt)
    norm = lv_ref[...] / (mean_lv + 1e-6)
    comb = jnp.clip(norm * 0.5 + up_ref[...], 0.0, 5.0)
    gate = jax.nn.sigmoid(gw_ref[0, 0] * comb + gb_ref[0, 0])
    out_ref[...] = hs_ref[...] + gate * agg_ref[...].astype(jnp.float32)


def _run_combine(hs2, agg, lv, up, lvs, gw, gb, nt_total, ts=512):
    nt, hidden = hs2.shape
    nsl = lvs.shape[0]
    grid = (nt // ts,)
    return pl.pallas_call(
        functools.partial(_combine_body, nt=nt_total),
        grid=grid,
        in_specs=[
            pl.BlockSpec((ts, hidden), lambda i: (i, 0)),
            pl.BlockSpec((ts, hidden), lambda i: (i, 0)),
            pl.BlockSpec((ts, 1), lambda i: (i, 0)),
            pl.BlockSpec((ts, 1), lambda i: (i, 0)),
            pl.BlockSpec((nsl, 1), lambda i: (0, 0), memory_space=pltpu.SMEM),
            pl.BlockSpec((1, 1), lambda i: (0, 0), memory_space=pltpu.SMEM),
            pl.BlockSpec((1, 1), lambda i: (0, 0), memory_space=pltpu.SMEM),
        ],
        out_specs=pl.BlockSpec((ts, hidden), lambda i: (i, 0)),
        out_shape=jax.ShapeDtypeStruct((nt, hidden), jnp.float32),
    )(hs2, agg, lv, up, lvs, gw, gb)


NSLICE = 1


def kernel(hidden_states, reliability_mask, Wq, Wr, aux_keys, aux_values,
           u_w1, u_b1, u_w2, u_b2, gate_w1, gate_bias):
    b, s, hidden = hidden_states.shape
    slots = aux_keys.shape[0]
    nt = b * s
    nts = nt // NSLICE
    hs2 = hidden_states.reshape(nt, hidden)

    R1, rk = _fold_weights(Wq, Wr, aux_keys)
    u_w1_bf = u_w1.astype(jnp.bfloat16)
    mask2 = reliability_mask.reshape(1, slots)
    ub1 = u_b1.reshape(1, -1)
    ub2 = u_b2.reshape(1, 1)
    aux_i32 = lax.bitcast_convert_type(
        aux_values.astype(jnp.bfloat16).reshape(slots, hidden // 2, 2),
        jnp.int32)
    sc_gather = _make_sc_gather(nts, hidden, slots)

    parts = []
    for i in range(NSLICE):
        hs_i = hs2[i * nts:(i + 1) * nts]
        w, idx, lv, up, lvs = _run_main(
            hs_i, R1, rk, mask2, u_w1_bf, ub1, u_w2, ub2)
        agg = sc_gather(idx.reshape(nts * TOPK), w.reshape(nts * TOPK),
                        aux_i32)
        parts.append((hs_i, agg, lv, up, lvs))

    lvs_all = jnp.concatenate([p[4] for p in parts], axis=0)
    gw = gate_w1.reshape(1, 1)
    gb = gate_bias.reshape(1, 1)
    outs = [
        _run_combine(hs_i, agg, lv, up, lvs_all, gw, gb, nt)
        for hs_i, agg, lv, up, _ in parts
    ]
    return jnp.concatenate(outs, axis=0).reshape(b, s, hidden)
